# baseline (speedup 1.0000x reference)
import functools

import jax
import jax.numpy as jnp
from jax import lax
from jax.experimental import pallas as pl
from jax.experimental.pallas import tpu as pltpu
from jax.experimental.pallas import tpu_sc as plsc

_N = 2000
_RPT = 63
_NWORKERS = 32
_NP = _RPT * _NWORKERS
_H = 128
_G = 16
_E = 32000
_CHUNK = 640
_GPC = _CHUNK // 16
_NCH = _E // _CHUNK

_BLK = 336
_MBLK = 400
_GH = _G * _H


def _build_adj(ei):
    mesh = plsc.VectorSubcoreMesh(core_axis_name="c", subcore_axis_name="s")

    ei_flat = ei.reshape(2 * _E)

    @functools.partial(
        pl.kernel,
        out_type=(jax.ShapeDtypeStruct((_NP * _N,), jnp.float32),
                  jax.ShapeDtypeStruct((_NWORKERS * 64,), jnp.float32)),
        mesh=mesh,
        compiler_params=pltpu.CompilerParams(needs_layout_passes=False),
        scratch_types=[
            pltpu.VMEM((_RPT * _N,), jnp.float32),
            pltpu.VMEM((64,), jnp.float32),
            pltpu.VMEM((_CHUNK,), jnp.int32),
            pltpu.VMEM((_CHUNK,), jnp.int32),
            pltpu.VMEM((_CHUNK,), jnp.int32),
            pltpu.VMEM((_CHUNK,), jnp.int32),
            pltpu.SemaphoreType.DMA,
            pltpu.SemaphoreType.DMA,
        ],
    )
    def k(ei_hbm, a_hbm, deg_hbm, acc_v, deg_v, src0, dst0, src1, dst1,
          sem0, sem1):
        wid = lax.axis_index("s") * 2 + lax.axis_index("c")
        lo = wid * _RPT
        io16 = lax.iota(jnp.int32, 16)
        zeros16 = jnp.zeros((16,), jnp.float32)
        ones16 = jnp.ones((16,), jnp.float32)

        bufs = ((src0, dst0), (src1, dst1))
        sems = (sem0, sem1)

        def start(b, cidx):
            off = cidx * _CHUNK
            pltpu.async_copy(ei_hbm.at[pl.ds(off, _CHUNK)], bufs[b][0],
                             sems[b])
            pltpu.async_copy(ei_hbm.at[pl.ds(_E + off, _CHUNK)], bufs[b][1],
                             sems[b])

        def wait(b):
            pltpu.make_async_copy(ei_hbm.at[pl.ds(0, _CHUNK)], bufs[b][0],
                                  sems[b]).wait()
            pltpu.make_async_copy(ei_hbm.at[pl.ds(0, _CHUNK)], bufs[b][1],
                                  sems[b]).wait()

        def process(b):
            src_v, dst_v = bufs[b]

            @plsc.parallel_loop(0, _GPC, 1, unroll=8)
            def grp(j):
                s = src_v[pl.ds(j * 16, 16)]
                d = dst_v[pl.ds(j * 16, 16)]
                mine = (d >= lo) & (d < lo + _RPT)
                occ, last = plsc.scan_count(d * _N + s, mask=mine)
                plsc.addupdate_scatter(acc_v, [(d - lo) * _N + s],
                                       occ.astype(jnp.float32),
                                       mask=last & mine)
                occd, lastd = plsc.scan_count(d, mask=mine)
                plsc.addupdate_scatter(deg_v, [d - lo],
                                       occd.astype(jnp.float32),
                                       mask=lastd & mine)

        start(0, 0)
        start(1, 1)

        @plsc.parallel_loop(0, _RPT * _N // 16, 1, unroll=8)
        def zero_blk(i):
            acc_v[pl.ds(i * 16, 16)] = zeros16

        for j in range(4):
            deg_v[pl.ds(j * 16, 16)] = ones16

        def pair(p, carry):
            more = p < _NCH // 2 - 1
            wait(0)
            process(0)

            @pl.when(more)
            def _():
                start(0, 2 * p + 2)
            wait(1)
            process(1)

            @pl.when(more)
            def _():
                start(1, 2 * p + 3)
            return carry
        lax.fori_loop(0, _NCH // 2, pair, 0)

        for j in range((_RPT + 15) // 16):
            r = j * 16 + io16
            gidx = lo + r
            plsc.addupdate_scatter(acc_v, [r * _N + gidx], ones16,
                                   mask=(r < _RPT) & (gidx < _N))

        pltpu.sync_copy(acc_v, a_hbm.at[pl.ds(lo * _N, _RPT * _N)])
        pltpu.sync_copy(deg_v, deg_hbm.at[pl.ds(wid * 64, 64)])

    a_flat, deg = k(ei_flat)
    return a_flat.reshape(_NP, _N), deg


def _ln_relu(t, g_ref, be_ref):
    mu = jnp.mean(t, axis=-1, keepdims=True)
    dlt = t - mu
    var = jnp.mean(dlt * dlt, axis=-1, keepdims=True)
    y = dlt * lax.rsqrt(var + 1e-5) * g_ref[...] + be_ref[...]
    return jnp.maximum(y, 0.0)


def _mega_body(x_ref, a_ref, dr_ref, dc_ref, w0_ref, w1_ref, w2_ref,
               b0_ref, g0_ref, e0_ref, b1_ref, g1_ref, e1_ref,
               b2_ref, g2_ref, e2_ref, out_ref,
               an_scr, ha_scr, hb_scr, r1_scr, r2_scr):
    i = pl.program_id(0)
    p = i // 5
    j = i % 5
    rows = pl.ds(j * _MBLK, _MBLK)

    @pl.when(p == 0)
    def _():
        w = w0_ref[...].astype(jnp.bfloat16)
        for g in range(_G):
            h = jnp.dot(x_ref[g], w, preferred_element_type=jnp.float32)
            ha_scr[rows, g * _H:(g + 1) * _H] = h.astype(jnp.bfloat16)

    @pl.when(p == 1)
    def _():
        dr = lax.rsqrt(jnp.maximum(dr_ref[...], 1e-12))
        dc = lax.rsqrt(jnp.maximum(dc_ref[...], 1e-12))
        an = (a_ref[...].astype(jnp.float32) * dr * dc).astype(jnp.bfloat16)
        an_scr[rows, :] = an
        acc = jnp.dot(an, ha_scr[...], preferred_element_type=jnp.float32)
        w = w1_ref[...]
        for g in range(_G):
            sl = slice(g * _H, (g + 1) * _H)
            y = _ln_relu(acc[:, sl] + b0_ref[...], g0_ref, e0_ref)
            r1_scr[rows, sl] = y.astype(jnp.bfloat16)
            hn = jnp.dot(y, w, preferred_element_type=jnp.float32)
            hb_scr[rows, sl] = hn.astype(jnp.bfloat16)

    @pl.when(p == 2)
    def _():
        acc = jnp.dot(an_scr[rows, :], hb_scr[...],
                      preferred_element_type=jnp.float32)
        w = w2_ref[...]
        for g in range(_G):
            sl = slice(g * _H, (g + 1) * _H)
            y = _ln_relu(acc[:, sl] + b1_ref[...], g1_ref, e1_ref)
            y = y + r1_scr[rows, sl].astype(jnp.float32)
            r2_scr[rows, sl] = y.astype(jnp.bfloat16)
            hn = jnp.dot(y, w, preferred_element_type=jnp.float32)
            ha_scr[rows, sl] = hn.astype(jnp.bfloat16)

    @pl.when(p == 3)
    def _():
        acc = jnp.dot(an_scr[rows, :], ha_scr[...],
                      preferred_element_type=jnp.float32)
        for g in range(_G):
            sl = slice(g * _H, (g + 1) * _H)
            y = _ln_relu(acc[:, sl] + b2_ref[...], g2_ref, e2_ref)
            y = y + r2_scr[rows, sl].astype(jnp.float32)
            out_ref[g] = y


def kernel(x, base_adj, base_edge_index, W_in, b_in, g_in, be_in,
           W_h1, b_h1, g_h1, be_h1, W_h2, b_h2, g_h2, be_h2):
    Bx, Tx, Nx, Cx = x.shape

    a_flat, deg = _build_adj(base_edge_index)
    A16 = a_flat.astype(jnp.bfloat16).reshape(_NP, _N)
    deg_r = deg.reshape(_NWORKERS, 64)[:, :_RPT].reshape(_NP, 1)
    deg_c = deg_r[:_N].reshape(1, _N)

    x16 = x.reshape(_G, _N, Cx).astype(jnp.bfloat16)

    def _c(v):
        return v.reshape(1, _H)

    out3 = pl.pallas_call(
        _mega_body,
        grid=(20,),
        in_specs=[
            pl.BlockSpec((_G, _MBLK, Cx),
                         lambda i: (0, jnp.where(i < 5, i, 0), 0)),
            pl.BlockSpec((_MBLK, _N),
                         lambda i: (jnp.where((i >= 5) & (i < 10), i - 5, 0),
                                    0)),
            pl.BlockSpec((_MBLK, 1),
                         lambda i: (jnp.where((i >= 5) & (i < 10), i - 5, 0),
                                    0)),
            pl.BlockSpec((1, _N), lambda i: (0, 0)),
            pl.BlockSpec((Cx, _H), lambda i: (0, 0)),
            pl.BlockSpec((_H, _H), lambda i: (0, 0)),
            pl.BlockSpec((_H, _H), lambda i: (0, 0)),
        ] + [pl.BlockSpec((1, _H), lambda i: (0, 0))] * 9,
        out_specs=pl.BlockSpec(
            (_G, _MBLK, _H),
            lambda i: (0, jnp.where(i >= 15, i - 15, 0), 0)),
        out_shape=jax.ShapeDtypeStruct((_G, _N, _H), jnp.float32),
        scratch_shapes=[
            pltpu.VMEM((_N, _N), jnp.bfloat16),
            pltpu.VMEM((_N, _GH), jnp.bfloat16),
            pltpu.VMEM((_N, _GH), jnp.bfloat16),
            pltpu.VMEM((_N, _GH), jnp.bfloat16),
            pltpu.VMEM((_N, _GH), jnp.bfloat16),
        ],
    )(x16, A16, deg_r, deg_c, W_in, W_h1, W_h2,
      _c(b_in), _c(g_in), _c(be_in), _c(b_h1), _c(g_h1), _c(be_h1),
      _c(b_h2), _c(g_h2), _c(be_h2))

    return out3.reshape(Bx, Tx, _N, _H)


# speedup vs baseline: 1.0143x; 1.0143x over previous
import functools

import jax
import jax.numpy as jnp
from jax import lax
from jax.experimental import pallas as pl
from jax.experimental.pallas import tpu as pltpu
from jax.experimental.pallas import tpu_sc as plsc

_N = 2000
_RPT = 63
_NWORKERS = 32
_NP = _RPT * _NWORKERS
_H = 128
_G = 16
_E = 32000
_CHUNK = 640
_GPC = _CHUNK // 16
_NCH = _E // _CHUNK

_BLK = 336
_MBLK = 400
_GH = _G * _H


def _build_adj(ei):
    mesh = plsc.VectorSubcoreMesh(core_axis_name="c", subcore_axis_name="s")

    ei_flat = ei.reshape(2 * _E)

    @functools.partial(
        pl.kernel,
        out_type=(jax.ShapeDtypeStruct((_NP * _N,), jnp.float32),
                  jax.ShapeDtypeStruct((_NWORKERS * 64,), jnp.float32)),
        mesh=mesh,
        compiler_params=pltpu.CompilerParams(needs_layout_passes=False),
        scratch_types=[
            pltpu.VMEM((_RPT * _N,), jnp.float32),
            pltpu.VMEM((64,), jnp.float32),
            pltpu.VMEM((_CHUNK,), jnp.int32),
            pltpu.VMEM((_CHUNK,), jnp.int32),
            pltpu.VMEM((_CHUNK,), jnp.int32),
            pltpu.VMEM((_CHUNK,), jnp.int32),
            pltpu.SemaphoreType.DMA,
            pltpu.SemaphoreType.DMA,
        ],
    )
    def k(ei_hbm, a_hbm, deg_hbm, acc_v, deg_v, src0, dst0, src1, dst1,
          sem0, sem1):
        wid = lax.axis_index("s") * 2 + lax.axis_index("c")
        lo = wid * _RPT
        io16 = lax.iota(jnp.int32, 16)
        zeros16 = jnp.zeros((16,), jnp.float32)
        ones16 = jnp.ones((16,), jnp.float32)

        bufs = ((src0, dst0), (src1, dst1))
        sems = (sem0, sem1)

        def start(b, cidx):
            off = cidx * _CHUNK
            pltpu.async_copy(ei_hbm.at[pl.ds(off, _CHUNK)], bufs[b][0],
                             sems[b])
            pltpu.async_copy(ei_hbm.at[pl.ds(_E + off, _CHUNK)], bufs[b][1],
                             sems[b])

        def wait(b):
            pltpu.make_async_copy(ei_hbm.at[pl.ds(0, _CHUNK)], bufs[b][0],
                                  sems[b]).wait()
            pltpu.make_async_copy(ei_hbm.at[pl.ds(0, _CHUNK)], bufs[b][1],
                                  sems[b]).wait()

        def process(b):
            src_v, dst_v = bufs[b]

            @plsc.parallel_loop(0, _GPC, 1, unroll=4)
            def grp(j):
                s = src_v[pl.ds(j * 16, 16)]
                d = dst_v[pl.ds(j * 16, 16)]
                mine = (d >= lo) & (d < lo + _RPT)
                occ, last = plsc.scan_count(d * _N + s, mask=mine)
                plsc.addupdate_scatter(acc_v, [(d - lo) * _N + s],
                                       occ.astype(jnp.float32),
                                       mask=last & mine)
                occd, lastd = plsc.scan_count(d, mask=mine)
                plsc.addupdate_scatter(deg_v, [d - lo],
                                       occd.astype(jnp.float32),
                                       mask=lastd & mine)

        start(0, 0)
        start(1, 1)

        @plsc.parallel_loop(0, _RPT * _N // 16, 1, unroll=8)
        def zero_blk(i):
            acc_v[pl.ds(i * 16, 16)] = zeros16

        for j in range(4):
            deg_v[pl.ds(j * 16, 16)] = ones16

        def pair(p, carry):
            more = p < _NCH // 2 - 1
            wait(0)
            process(0)

            @pl.when(more)
            def _():
                start(0, 2 * p + 2)
            wait(1)
            process(1)

            @pl.when(more)
            def _():
                start(1, 2 * p + 3)
            return carry
        lax.fori_loop(0, _NCH // 2, pair, 0)

        for j in range((_RPT + 15) // 16):
            r = j * 16 + io16
            gidx = lo + r
            plsc.addupdate_scatter(acc_v, [r * _N + gidx], ones16,
                                   mask=(r < _RPT) & (gidx < _N))

        pltpu.sync_copy(acc_v, a_hbm.at[pl.ds(lo * _N, _RPT * _N)])
        pltpu.sync_copy(deg_v, deg_hbm.at[pl.ds(wid * 64, 64)])

    a_flat, deg = k(ei_flat)
    return a_flat.reshape(_NP, _N), deg


def _ln_relu(t, g_ref, be_ref):
    mu = jnp.mean(t, axis=-1, keepdims=True)
    dlt = t - mu
    var = jnp.mean(dlt * dlt, axis=-1, keepdims=True)
    y = dlt * lax.rsqrt(var + 1e-5) * g_ref[...] + be_ref[...]
    return jnp.maximum(y, 0.0)


def _mega_body(x_ref, a_ref, dr_ref, dc_ref, w0_ref, w1_ref, w2_ref,
               b0_ref, g0_ref, e0_ref, b1_ref, g1_ref, e1_ref,
               b2_ref, g2_ref, e2_ref, out_ref,
               an_scr, ha_scr, hb_scr, r1_scr, r2_scr):
    i = pl.program_id(0)
    p = i // 5
    j = i % 5
    rows = pl.ds(j * _MBLK, _MBLK)

    @pl.when(p == 0)
    def _():
        w = w0_ref[...].astype(jnp.bfloat16)
        for g in range(_G):
            h = jnp.dot(x_ref[g], w, preferred_element_type=jnp.float32)
            ha_scr[rows, g * _H:(g + 1) * _H] = h.astype(jnp.bfloat16)

    @pl.when(p == 1)
    def _():
        dr = lax.rsqrt(jnp.maximum(dr_ref[...], 1e-12))
        dc = lax.rsqrt(jnp.maximum(dc_ref[...], 1e-12))
        an = (a_ref[...].astype(jnp.float32) * dr * dc).astype(jnp.bfloat16)
        an_scr[rows, :] = an
        acc = jnp.dot(an, ha_scr[...], preferred_element_type=jnp.float32)
        w = w1_ref[...]
        for g in range(_G):
            sl = slice(g * _H, (g + 1) * _H)
            y = _ln_relu(acc[:, sl] + b0_ref[...], g0_ref, e0_ref)
            r1_scr[rows, sl] = y.astype(jnp.bfloat16)
            hn = jnp.dot(y, w, preferred_element_type=jnp.float32)
            hb_scr[rows, sl] = hn.astype(jnp.bfloat16)

    @pl.when(p == 2)
    def _():
        acc = jnp.dot(an_scr[rows, :], hb_scr[...],
                      preferred_element_type=jnp.float32)
        w = w2_ref[...]
        for g in range(_G):
            sl = slice(g * _H, (g + 1) * _H)
            y = _ln_relu(acc[:, sl] + b1_ref[...], g1_ref, e1_ref)
            y = y + r1_scr[rows, sl].astype(jnp.float32)
            r2_scr[rows, sl] = y.astype(jnp.bfloat16)
            hn = jnp.dot(y, w, preferred_element_type=jnp.float32)
            ha_scr[rows, sl] = hn.astype(jnp.bfloat16)

    @pl.when(p == 3)
    def _():
        acc = jnp.dot(an_scr[rows, :], ha_scr[...],
                      preferred_element_type=jnp.float32)
        for g in range(_G):
            sl = slice(g * _H, (g + 1) * _H)
            y = _ln_relu(acc[:, sl] + b2_ref[...], g2_ref, e2_ref)
            y = y + r2_scr[rows, sl].astype(jnp.float32)
            out_ref[g] = y


def kernel(x, base_adj, base_edge_index, W_in, b_in, g_in, be_in,
           W_h1, b_h1, g_h1, be_h1, W_h2, b_h2, g_h2, be_h2):
    Bx, Tx, Nx, Cx = x.shape

    a_flat, deg = _build_adj(base_edge_index)
    A16 = a_flat.astype(jnp.bfloat16).reshape(_NP, _N)
    deg_r = deg.reshape(_NWORKERS, 64)[:, :_RPT].reshape(_NP, 1)
    deg_c = deg_r[:_N].reshape(1, _N)

    x16 = x.reshape(_G, _N, Cx).astype(jnp.bfloat16)

    def _c(v):
        return v.reshape(1, _H)

    out3 = pl.pallas_call(
        _mega_body,
        grid=(20,),
        in_specs=[
            pl.BlockSpec((_G, _MBLK, Cx),
                         lambda i: (0, jnp.where(i < 5, i, 0), 0)),
            pl.BlockSpec((_MBLK, _N),
                         lambda i: (jnp.where((i >= 5) & (i < 10), i - 5, 0),
                                    0)),
            pl.BlockSpec((_MBLK, 1),
                         lambda i: (jnp.where((i >= 5) & (i < 10), i - 5, 0),
                                    0)),
            pl.BlockSpec((1, _N), lambda i: (0, 0)),
            pl.BlockSpec((Cx, _H), lambda i: (0, 0)),
            pl.BlockSpec((_H, _H), lambda i: (0, 0)),
            pl.BlockSpec((_H, _H), lambda i: (0, 0)),
        ] + [pl.BlockSpec((1, _H), lambda i: (0, 0))] * 9,
        out_specs=pl.BlockSpec(
            (_G, _MBLK, _H),
            lambda i: (0, jnp.where(i >= 15, i - 15, 0), 0)),
        out_shape=jax.ShapeDtypeStruct((_G, _N, _H), jnp.float32),
        scratch_shapes=[
            pltpu.VMEM((_N, _N), jnp.bfloat16),
            pltpu.VMEM((_N, _GH), jnp.bfloat16),
            pltpu.VMEM((_N, _GH), jnp.bfloat16),
            pltpu.VMEM((_N, _GH), jnp.bfloat16),
            pltpu.VMEM((_N, _GH), jnp.bfloat16),
        ],
    )(x16, A16, deg_r, deg_c, W_in, W_h1, W_h2,
      _c(b_in), _c(g_in), _c(be_in), _c(b_h1), _c(g_h1), _c(be_h1),
      _c(b_h2), _c(g_h2), _c(be_h2))

    return out3.reshape(Bx, Tx, _N, _H)
